# baseline (speedup 1.0000x reference)
import functools

import jax, jax.numpy as jnp
from jax import lax
from jax.experimental import pallas as pl
from jax.experimental.pallas import tpu as pltpu
from jax.experimental.pallas import tpu_sc as plsc

_NC = 64
_C = 8192

_N = 100000
_FULL = 99968
_CH = 1408
_NCHUNKS = _FULL // _CH
_NW = 32


def _tc_body(idx_ref, a_ref):
    idx = idx_ref[0]
    iota = lax.broadcasted_iota(jnp.int32, (_NC, _C), 0)
    a_ref[...] = (iota == idx).astype(jnp.float32)


_NT = _CH // 128


def _sc_body(idx_hbm, out_hbm, idx_v, buf, sem):
    wid = lax.axis_index("s") * 2 + lax.axis_index("c")

    def zero_tile(i, _):
        jhi = i // (_NT * 8)
        rem = i % (_NT * 8)
        t = rem // 8
        jlo = rem % 8
        def zero_vec(v, _):
            buf[jhi, t, jlo, pl.ds(v * 16, 16)] = jnp.zeros((16,), jnp.float32)
            return 0
        return lax.fori_loop(0, 8, zero_vec, 0)

    lax.fori_loop(0, 8 * _NT * 8, zero_tile, 0)

    ones = jnp.full((16,), 1.0, jnp.float32)
    zeros = jnp.zeros((16,), jnp.float32)
    lane16 = lax.iota(jnp.int32, 16)

    def do_chunk(k, _):
        c = wid + _NW * k

        @pl.when(c < _NCHUNKS)
        def _():
            c0 = c * _CH
            pltpu.sync_copy(idx_hbm.at[pl.ds(c0, _CH)], idx_v)

            def scat_with(val):
                def scat(v, _):
                    iv = idx_v[pl.ds(v * 16, 16)]
                    rloc = lane16 + v * 16
                    jhi = lax.shift_right_logical(iv, 3)
                    jlo = lax.bitwise_and(iv, 7)
                    t = lax.shift_right_logical(rloc, 7)
                    ln = lax.bitwise_and(rloc, 127)
                    plsc.store_scatter(buf, [jhi, t, jlo, ln], val)
                    return 0
                return scat

            lax.fori_loop(0, _CH // 16, scat_with(ones), 0)
            copies = [
                pltpu.async_copy(
                    buf.at[jhi, t],
                    out_hbm.at[pl.ds(jhi * 8, 8), pl.ds(c0 + t * 128, 128)],
                    sem,
                )
                for jhi in range(8)
                for t in range(_NT)
            ]
            for h in copies:
                h.wait()
            lax.fori_loop(0, _CH // 16, scat_with(zeros), 0)

        return 0

    lax.fori_loop(0, 3, do_chunk, 0)


def _tail_body(idx_ref, b_in_ref, b_ref):
    del b_in_ref
    idx = idx_ref[0]
    iota = lax.broadcasted_iota(jnp.int32, (_NC, 128), 0)
    b_ref[...] = (iota == idx).astype(jnp.float32)


def kernel(species_index, pos):
    n = species_index.shape[0]
    idx = species_index.astype(jnp.int32)
    g = (n + _C - 1) // _C
    idx3 = idx.reshape(1, 1, n)

    a = pl.pallas_call(
        _tc_body,
        grid=(g,),
        in_specs=[pl.BlockSpec((1, 1, _C), lambda i: (0, 0, i))],
        out_specs=pl.BlockSpec((_NC, _C), lambda i: (0, i)),
        out_shape=jax.ShapeDtypeStruct((_NC, n), jnp.float32),
    )(idx3)

    sc = pl.kernel(
        _sc_body,
        out_type=jax.ShapeDtypeStruct((_NC, n), jnp.float32),
        mesh=plsc.VectorSubcoreMesh(core_axis_name="c", subcore_axis_name="s"),
        scratch_types=[
            pltpu.VMEM((_CH,), jnp.int32),
            pltpu.VMEM((8, _NT, 8, 128), jnp.float32),
            pltpu.SemaphoreType.DMA,
        ],
        compiler_params=pltpu.CompilerParams(use_tc_tiling_on_sc=True, needs_layout_passes=False),
    )
    b_sc = sc(idx)

    tail = n - _FULL
    idx_tail = jnp.pad(lax.slice(idx, (_FULL,), (n,)), (0, 128 - tail))
    idx_tail = idx_tail.reshape(1, 1, 128)
    b = pl.pallas_call(
        _tail_body,
        grid=(1,),
        in_specs=[
            pl.BlockSpec((1, 1, 128), lambda i: (0, 0, 0)),
            pl.BlockSpec(memory_space=pl.ANY),
        ],
        out_specs=pl.BlockSpec((_NC, 128), lambda i: (0, _FULL // 128)),
        out_shape=jax.ShapeDtypeStruct((_NC, n), jnp.float32),
        input_output_aliases={1: 0},
    )(idx_tail, b_sc)

    return a.T, b.T


# speedup vs baseline: 2.2691x; 2.2691x over previous
import jax, jax.numpy as jnp
from jax import lax
from jax.experimental import pallas as pl

_NC = 64
_C = 8192


def _body(idx_ref, a_ref, b_ref):
    idx = idx_ref[0]
    iota = lax.broadcasted_iota(jnp.int32, (_NC, _C), 0)
    oh = (iota == idx).astype(jnp.float32)
    a_ref[...] = oh
    b_ref[...] = oh


def kernel(species_index, pos):
    n = species_index.shape[0]
    g = (n + _C - 1) // _C
    idx3 = species_index.astype(jnp.int32).reshape(1, 1, n)
    spec = pl.BlockSpec((_NC, _C), lambda i: (0, i))
    a, b = pl.pallas_call(
        _body,
        grid=(g,),
        in_specs=[pl.BlockSpec((1, 1, _C), lambda i: (0, 0, i))],
        out_specs=[spec, spec],
        out_shape=[jax.ShapeDtypeStruct((_NC, n), jnp.float32)] * 2,
    )(idx3)
    return a.T, b.T
